# baseline (speedup 1.0000x reference)
import functools

import jax
import jax.numpy as jnp
from jax import lax
from jax.experimental import pallas as pl
from jax.experimental.pallas import tpu as pltpu
from jax.experimental.pallas import tpu_sc as plsc

_NC = 2
_NS = 16
_L = 16
_NW = _NC * _NS


def _sc_body(x_hbm, emb_hbm, out_hbm, buf, embrow, in_sem, out_sem):
    B, T, S, D = x_hbm.shape
    SUB = 4
    SR = S // SUB
    CPW = (B * T) // _NW
    NQ = CPW * SUB

    wid = lax.axis_index("s") * _NC + lax.axis_index("c")
    q0 = wid * CPW * SUB

    def coords(q):
        cs = (q0 + q) // SUB
        sub = lax.rem(q0 + q, SUB)
        b = cs // T
        t = lax.rem(cs, T)
        return b, t, sub

    def in_copy(q, slot):
        b, t, sub = coords(q)
        return pltpu.make_async_copy(
            x_hbm.at[b, t, pl.ds(sub * SR, SR)], buf.at[slot], in_sem.at[slot]
        )

    def out_copy(q, slot):
        b, t, sub = coords(q)
        return pltpu.make_async_copy(
            buf.at[slot], out_hbm.at[b, t, pl.ds(sub * SR, SR)], out_sem.at[slot]
        )

    in_copy(0, 0).start()
    in_copy(1, 1).start()

    def step(q, carry):
        slot = lax.rem(q, 3)
        b, t, sub = coords(q)

        @pl.when(sub == 0)
        def _():
            pltpu.sync_copy(emb_hbm.at[t], embrow)

        in_copy(q, slot).wait()

        def jloop(j, c):
            ev = embrow[pl.ds(j * _L, _L)]

            def rloop(r, c2):
                plsc.addupdate(buf.at[slot, r, pl.ds(j * _L, _L)], ev)
                return c2

            return lax.fori_loop(0, SR, rloop, c, unroll=4)

        lax.fori_loop(0, D // _L, jloop, 0)

        out_copy(q, slot).start()

        nq = q + 2
        nslot = lax.rem(nq, 3)

        @pl.when(nq < NQ)
        def _():
            @pl.when(q >= 1)
            def _():
                out_copy(q - 1, nslot).wait()

            in_copy(nq, nslot).start()

        return carry

    lax.fori_loop(0, NQ, step, 0)

    out_copy(NQ - 3, lax.rem(NQ - 3, 3)).wait()
    out_copy(NQ - 2, lax.rem(NQ - 2, 3)).wait()
    out_copy(NQ - 1, lax.rem(NQ - 1, 3)).wait()


def kernel(x, emb):
    B, T, S, D = x.shape
    SR = S // 4
    mesh = plsc.VectorSubcoreMesh(core_axis_name="c", subcore_axis_name="s")
    f = functools.partial(
        pl.kernel,
        mesh=mesh,
        compiler_params=pltpu.CompilerParams(use_tc_tiling_on_sc=False),
        out_type=jax.ShapeDtypeStruct((B, T, S, D), jnp.float32),
        scratch_types=[
            pltpu.VMEM((3, SR, D), jnp.float32),
            pltpu.VMEM((D,), jnp.float32),
            pltpu.SemaphoreType.DMA((3,)),
            pltpu.SemaphoreType.DMA((3,)),
        ],
    )(_sc_body)
    return f(x, emb)


# speedup vs baseline: 1.7733x; 1.7733x over previous
import functools

import jax
import jax.numpy as jnp
from jax import lax
from jax.experimental import pallas as pl
from jax.experimental.pallas import tpu as pltpu
from jax.experimental.pallas import tpu_sc as plsc

_NC = 2
_NS = 16
_L = 16
_NW = _NC * _NS
_DB = 128


def _sc_body(x_hbm, emb_hbm, out_hbm, buf, embrow, in_sem, out_sem):
    B, T, S, D = x_hbm.shape
    ND = D // _DB
    NQ = (B * T * ND) // _NW

    wid = lax.axis_index("s") * _NC + lax.axis_index("c")
    q0 = wid * NQ

    def coords(q):
        g = q0 + q
        cs = g // ND
        dj = lax.rem(g, ND)
        b = cs // T
        t = lax.rem(cs, T)
        return b, t, dj

    def in_copy(q, slot):
        b, t, dj = coords(q)
        return pltpu.make_async_copy(
            x_hbm.at[b, t, :, pl.ds(dj * _DB, _DB)], buf.at[slot],
            in_sem.at[slot],
        )

    def out_copy(q, slot):
        b, t, dj = coords(q)
        return pltpu.make_async_copy(
            buf.at[slot], out_hbm.at[b, t, :, pl.ds(dj * _DB, _DB)],
            out_sem.at[slot],
        )

    in_copy(0, 0).start()
    in_copy(1, 1).start()

    def step(q, carry):
        slot = lax.rem(q, 3)
        b, t, dj = coords(q)

        @pl.when(lax.rem(q0 + q, ND) == 0)
        def _():
            pltpu.sync_copy(emb_hbm.at[t], embrow)

        in_copy(q, slot).wait()

        def jloop(j, c):
            ev = embrow[0, pl.ds(dj * _DB + j * _L, _L)]

            def rloop(r, c2):
                plsc.addupdate(buf.at[slot, r, pl.ds(j * _L, _L)], ev)
                return c2

            return lax.fori_loop(0, S, rloop, c, unroll=4)

        lax.fori_loop(0, _DB // _L, jloop, 0)

        out_copy(q, slot).start()

        nq = q + 2
        nslot = lax.rem(nq, 3)

        @pl.when(nq < NQ)
        def _():
            @pl.when(q >= 1)
            def _():
                out_copy(q - 1, nslot).wait()

            in_copy(nq, nslot).start()

        return carry

    lax.fori_loop(0, NQ, step, 0)

    out_copy(NQ - 3, lax.rem(NQ - 3, 3)).wait()
    out_copy(NQ - 2, lax.rem(NQ - 2, 3)).wait()
    out_copy(NQ - 1, lax.rem(NQ - 1, 3)).wait()


def kernel(x, emb):
    B, T, S, D = x.shape
    emb3 = emb.reshape(T, 1, D)
    mesh = plsc.VectorSubcoreMesh(core_axis_name="c", subcore_axis_name="s")
    f = functools.partial(
        pl.kernel,
        mesh=mesh,
        out_type=jax.ShapeDtypeStruct((B, T, S, D), jnp.float32),
        scratch_types=[
            pltpu.VMEM((3, S, _DB), jnp.float32),
            pltpu.VMEM((1, D), jnp.float32),
            pltpu.SemaphoreType.DMA((3,)),
            pltpu.SemaphoreType.DMA((3,)),
        ],
    )(_sc_body)
    return f(x, emb3)


# speedup vs baseline: 1.9417x; 1.0949x over previous
import jax
import jax.numpy as jnp
from jax.experimental import pallas as pl


def _add_body(x_ref, emb_ref, o_ref):
    o_ref[...] = x_ref[...] + emb_ref[...]


def kernel(x, emb):
    B, T, S, D = x.shape
    TB = 8
    emb3 = emb.reshape(T, 1, D)
    return pl.pallas_call(
        _add_body,
        grid=(B, T // TB),
        in_specs=[
            pl.BlockSpec((1, TB, S, D), lambda i, j: (i, j, 0, 0)),
            pl.BlockSpec((TB, 1, D), lambda i, j: (j, 0, 0)),
        ],
        out_specs=pl.BlockSpec((1, TB, S, D), lambda i, j: (i, j, 0, 0)),
        out_shape=jax.ShapeDtypeStruct(x.shape, x.dtype),
    )(x, emb3)


# speedup vs baseline: 1.9492x; 1.0038x over previous
import jax
import jax.numpy as jnp
from jax.experimental import pallas as pl
from jax.experimental.pallas import tpu as pltpu


def _add_body(x_ref, emb_ref, o_ref):
    o_ref[...] = x_ref[...] + emb_ref[...]


def kernel(x, emb):
    B, T, S, D = x.shape
    TB = 16
    emb3 = emb.reshape(T, 1, D)
    return pl.pallas_call(
        _add_body,
        grid=(B, T // TB),
        in_specs=[
            pl.BlockSpec((1, TB, S, D), lambda i, j: (i, j, 0, 0)),
            pl.BlockSpec((TB, 1, D), lambda i, j: (j, 0, 0)),
        ],
        out_specs=pl.BlockSpec((1, TB, S, D), lambda i, j: (i, j, 0, 0)),
        out_shape=jax.ShapeDtypeStruct(x.shape, x.dtype),
        compiler_params=pltpu.CompilerParams(
            dimension_semantics=("parallel", "parallel"),
        ),
    )(x, emb3)
